# baseline (speedup 1.0000x reference)
import dataclasses
import functools

import jax
import jax.numpy as jnp
from jax import lax
from jax.experimental import pallas as pl
from jax.experimental.pallas import tpu as pltpu
from jax.experimental.pallas import tpu_sc as plsc

_EPS = 1e-5
_B = 16384
_D = 32
_NC = 2
_NS = 16
_NW = _NC * _NS
_PPW = _B // _NW
_RPW = 2 * _PPW
_GCHUNK = 128
_L = 16


def _sc_distance_arg(flat_idx, emb_table):
    mesh = plsc.VectorSubcoreMesh(core_axis_name="c", subcore_axis_name="s")
    cp = pltpu.CompilerParams()
    for field, val in (("needs_layout_passes", False),
                       ("use_tc_tiling_on_sc", False)):
        if field in pltpu.CompilerParams.__dataclass_fields__:
            cp = dataclasses.replace(cp, **{field: val})

    @functools.partial(
        pl.kernel,
        mesh=mesh,
        compiler_params=cp,
        out_type=jax.ShapeDtypeStruct((_B,), jnp.float32),
        scratch_types=[
            pltpu.VMEM((_RPW,), jnp.int32),
            pltpu.VMEM((_RPW, _D), jnp.float32),
            pltpu.VMEM((_PPW,), jnp.float32),
            pltpu.SemaphoreType.DMA,
        ],
    )
    def sc_kernel(idx_hbm, table_hbm, out_hbm, idx_v, rows_v, x_v, sem):
        wid = lax.axis_index("s") * _NC + lax.axis_index("c")
        row_base = wid * _RPW
        pltpu.sync_copy(idx_hbm.at[pl.ds(row_base, _RPW)], idx_v)

        copies = []
        for g in range(_RPW // _GCHUNK):
            copies.append(
                pltpu.async_copy(
                    table_hbm.at[idx_v.at[pl.ds(g * _GCHUNK, _GCHUNK)]],
                    rows_v.at[pl.ds(g * _GCHUNK, _GCHUNK)],
                    sem,
                )
            )
        for c in copies:
            c.wait()

        lane = lax.iota(jnp.int32, _L)

        @pl.loop(0, _PPW // _L)
        def _(g):
            sq_acc = un_acc = vn_acc = None
            for k in range(_L):
                r = 2 * (g * _L + k)
                u0 = rows_v[r, pl.ds(0, _L)]
                u1 = rows_v[r, pl.ds(_L, _L)]
                v0 = rows_v[r + 1, pl.ds(0, _L)]
                v1 = rows_v[r + 1, pl.ds(_L, _L)]
                d0 = u0 - v0
                d1 = u1 - v1
                sq = jnp.full((_L,), jnp.sum(d0 * d0 + d1 * d1))
                un = jnp.full((_L,), jnp.sum(u0 * u0 + u1 * u1))
                vn = jnp.full((_L,), jnp.sum(v0 * v0 + v1 * v1))
                if k == 0:
                    sq_acc, un_acc, vn_acc = sq, un, vn
                else:
                    m = lane == k
                    sq_acc = jnp.where(m, sq, sq_acc)
                    un_acc = jnp.where(m, un, un_acc)
                    vn_acc = jnp.where(m, vn, vn_acc)
            un_acc = jnp.minimum(jnp.maximum(un_acc, 0.0), 1.0 - _EPS)
            vn_acc = jnp.minimum(jnp.maximum(vn_acc, 0.0), 1.0 - _EPS)
            x_v[pl.ds(g * _L, _L)] = 1.0 + 2.0 * sq_acc / (
                (1.0 - un_acc) * (1.0 - vn_acc)
            )

        pltpu.sync_copy(x_v, out_hbm.at[pl.ds(wid * _PPW, _PPW)])

    return sc_kernel(flat_idx, emb_table)


def _tc_arccosh(x2d):
    def body(x_ref, o_ref):
        x = jnp.maximum(x_ref[...], 1.0 + _EPS)
        o_ref[...] = jnp.log(x + jnp.sqrt(x * x - 1.0))

    return pl.pallas_call(
        body,
        out_shape=jax.ShapeDtypeStruct(x2d.shape, jnp.float32),
    )(x2d)


@jax.jit
def kernel(input_triplet, emb_table):
    flat_idx = input_triplet.reshape(-1)
    x = _sc_distance_arg(flat_idx, emb_table)
    dist = _tc_arccosh(x.reshape(_B // 128, 128))
    return dist.reshape(_B)
